# Initial kernel scaffold; baseline (speedup 1.0000x reference)
import jax
import jax.numpy as jnp
from jax.experimental import pallas as pl


def kernel(x, edge_index, W1_0, W1_1, b1, W2_0, W2_1, b2, W3_0, W3_1, b3):
    raise NotImplementedError("write your pallas kernel here")


# speedup vs baseline: 11.0934x; 11.0934x over previous
import functools

import jax
import jax.numpy as jnp
from jax import lax
from jax.experimental import pallas as pl
from jax.experimental.pallas import tpu as pltpu
from jax.experimental.pallas import tpu_sc as plsc

N = 10000
E = 160000
CHUNK = 128
NCHUNK = E // CHUNK
NW = 32
BASE_CH = NCHUNK // NW
EXTRA_CH = NCHUNK - BASE_CH * NW
RPT = 624


def _mesh():
    return plsc.VectorSubcoreMesh(core_axis_name="c", subcore_axis_name="s")


def _worker_span(w):
    start = w * BASE_CH + jnp.minimum(w, EXTRA_CH)
    cnt = BASE_CH + jnp.where(w < EXTRA_CH, 1, 0)
    return start, cnt


@functools.cache
def _make_deg():
    @functools.partial(
        pl.kernel,
        out_type=jax.ShapeDtypeStruct((2 * N,), jnp.float32),
        mesh=_mesh(),
        scratch_types=[
            pltpu.VMEM((CHUNK,), jnp.int32),
            pltpu.VMEM((CHUNK,), jnp.float32),
            pltpu.VMEM((RPT,), jnp.float32),
            pltpu.VMEM_SHARED((N,), jnp.float32),
        ],
    )
    def deg_kernel(src_hbm, out_hbm, sidx, ones_v, zbuf, acc):
        c = lax.axis_index("c")
        s = lax.axis_index("s")
        w = c * 16 + s
        one16 = jnp.full((16,), 1.0, jnp.float32)
        zero16 = jnp.zeros((16,), jnp.float32)
        for j in range(CHUNK // 16):
            ones_v[pl.ds(16 * j, 16)] = one16
        for j in range(RPT // 16):
            zbuf[pl.ds(16 * j, 16)] = zero16
        base = pl.multiple_of(s * RPT, 8)
        pltpu.sync_copy(zbuf, acc.at[pl.ds(base, RPT)])

        @pl.when(s == 15)
        def _():
            pltpu.sync_copy(zbuf.at[pl.ds(0, 16)], acc.at[pl.ds(N - 16, 16)])

        plsc.subcore_barrier()
        start, cnt = _worker_span(w)

        def body(t, carry):
            e0 = pl.multiple_of((start + t) * CHUNK, CHUNK)
            pltpu.sync_copy(src_hbm.at[pl.ds(e0, CHUNK)], sidx)
            pltpu.sync_copy(ones_v, acc.at[sidx], add=True)
            return carry

        lax.fori_loop(0, cnt, body, 0)
        plsc.subcore_barrier()
        ob = pl.multiple_of(c * N + base, 8)
        pltpu.sync_copy(acc.at[pl.ds(base, RPT)], zbuf)
        pltpu.sync_copy(zbuf, out_hbm.at[pl.ds(ob, RPT)])

        @pl.when(s == 15)
        def _():
            pltpu.sync_copy(acc.at[pl.ds(N - 16, 16)], zbuf.at[pl.ds(0, 16)])
            pltpu.sync_copy(zbuf.at[pl.ds(0, 16)],
                            out_hbm.at[pl.ds(c * N + N - 16, 16)])

    return deg_kernel


@functools.cache
def _make_prop(F):
    @functools.partial(
        pl.kernel,
        out_type=jax.ShapeDtypeStruct((2 * N, F), jnp.float32),
        mesh=_mesh(),
        scratch_types=[
            pltpu.VMEM((CHUNK,), jnp.int32),
            pltpu.VMEM((CHUNK,), jnp.int32),
            pltpu.VMEM((CHUNK, F), jnp.float32),
            pltpu.VMEM((16, F), jnp.float32),
            pltpu.VMEM_SHARED((N, F), jnp.float32),
        ],
        compiler_params=pltpu.CompilerParams(use_tc_tiling_on_sc=False),
    )
    def prop_kernel(y_hbm, src_hbm, dst_hbm, out_hbm, sidx, didx, rows, zbuf,
                    acc):
        c = lax.axis_index("c")
        s = lax.axis_index("s")
        w = c * 16 + s
        zero16 = jnp.zeros((16,), jnp.float32)
        for r in range(16):
            for j in range(F // 16):
                zbuf[r, pl.ds(16 * j, 16)] = zero16
        base = pl.multiple_of(s * RPT, 8)

        def zloop(i, carry):
            pltpu.sync_copy(zbuf, acc.at[pl.ds(base + 16 * i, 16)])
            return carry

        lax.fori_loop(0, RPT // 16, zloop, 0)

        @pl.when(s == 15)
        def _():
            pltpu.sync_copy(zbuf, acc.at[pl.ds(N - 16, 16)])

        plsc.subcore_barrier()
        start, cnt = _worker_span(w)

        def body(t, carry):
            e0 = pl.multiple_of((start + t) * CHUNK, CHUNK)
            pltpu.sync_copy(src_hbm.at[pl.ds(e0, CHUNK)], sidx)
            pltpu.sync_copy(dst_hbm.at[pl.ds(e0, CHUNK)], didx)
            pltpu.sync_copy(y_hbm.at[sidx], rows)
            pltpu.sync_copy(rows, acc.at[didx], add=True)
            return carry

        lax.fori_loop(0, cnt, body, 0)
        plsc.subcore_barrier()
        ob = pl.multiple_of(c * N + base, 8)
        for i in range(RPT // CHUNK):
            pltpu.sync_copy(acc.at[pl.ds(base + i * CHUNK, CHUNK)], rows)
            pltpu.sync_copy(rows, out_hbm.at[pl.ds(ob + i * CHUNK, CHUNK)])
        tail0 = (RPT // CHUNK) * CHUNK
        for j in range((RPT - tail0) // 16):
            pltpu.sync_copy(acc.at[pl.ds(base + tail0 + j * 16, 16)], zbuf)
            pltpu.sync_copy(zbuf, out_hbm.at[pl.ds(ob + tail0 + j * 16, 16)])

        @pl.when(s == 15)
        def _():
            pltpu.sync_copy(acc.at[pl.ds(N - 16, 16)], zbuf)
            pltpu.sync_copy(zbuf, out_hbm.at[pl.ds(c * N + N - 16, 16)])

    return prop_kernel


def _prep_body(degp_ref, x_ref, dinv_ref, xs_ref):
    dp = degp_ref[...]
    ones2 = jnp.ones((2, 1), jnp.float32)
    deg = lax.dot_general(dp, ones2, (((0,), (0,)), ((), ())),
                          preferred_element_type=jnp.float32)
    safe = jnp.where(deg > 0, deg, 1.0)
    dinv = jnp.where(deg > 0, lax.rsqrt(safe), 0.0)
    dinv_ref[...] = dinv
    xs = x_ref[...] * dinv
    xs_ref[...] = jnp.concatenate(
        [xs, jnp.zeros((N, 64 - 58), jnp.float32)], axis=1)


def _l1_body(x_ref, g1a_ref, g1b_ref, dinv_ref, w10_ref, w11_ref, b1_ref,
             w21_ref, h1_ref, y2_ref):
    dinv = dinv_ref[...]
    tx1 = -(g1a_ref[...] + g1b_ref[...])[:, :58] * dinv
    h1 = jnp.dot(x_ref[...], w10_ref[...], preferred_element_type=jnp.float32)
    h1 += jnp.dot(tx1, w11_ref[...], preferred_element_type=jnp.float32)
    h1 = jnp.maximum(h1 + b1_ref[...], 0.0)
    h1_ref[...] = h1
    y2 = jnp.dot(h1, w21_ref[...], preferred_element_type=jnp.float32) * dinv
    y2_ref[...] = jnp.concatenate(
        [y2, jnp.zeros((N, 112 - 100), jnp.float32)], axis=1)


def _l2_body(h1_ref, g2a_ref, g2b_ref, dinv_ref, w20_ref, b2_ref, w31_ref,
             h2_ref, y3_ref):
    dinv = dinv_ref[...]
    tx2w = -(g2a_ref[...] + g2b_ref[...])[:, :100] * dinv
    h2 = jnp.dot(h1_ref[...], w20_ref[...],
                 preferred_element_type=jnp.float32)
    h2 = jnp.maximum(h2 + tx2w + b2_ref[...], 0.0)
    h2_ref[...] = h2
    y3 = jnp.dot(h2, w31_ref[...], preferred_element_type=jnp.float32) * dinv
    y3_ref[...] = jnp.concatenate(
        [y3, jnp.zeros((N, 15), jnp.float32)], axis=1)


def _final_body(h2_ref, g3a_ref, g3b_ref, dinv_ref, w30_ref, b3_ref, out_ref):
    tx3w = -(g3a_ref[...] + g3b_ref[...])[:, :1] * dinv_ref[...]
    out = jnp.dot(h2_ref[...], w30_ref[...],
                  preferred_element_type=jnp.float32)
    out_ref[...] = out + tx3w + b3_ref[...]


def _sds(shape):
    return jax.ShapeDtypeStruct(shape, jnp.float32)


def kernel(x, edge_index, W1_0, W1_1, b1, W2_0, W2_1, b2, W3_0, W3_1, b3):
    src = edge_index[0]
    dst = edge_index[1]

    degp = _make_deg()(src).reshape(2, N)

    dinv, xs = pl.pallas_call(
        _prep_body, out_shape=(_sds((N, 1)), _sds((N, 64))))(degp, x)

    g1 = _make_prop(64)(xs, src, dst)
    h1, y2 = pl.pallas_call(
        _l1_body, out_shape=(_sds((N, 300)), _sds((N, 112))))(
            x, g1[:N], g1[N:], dinv, W1_0, W1_1, b1.reshape(1, -1), W2_1)

    g2 = _make_prop(112)(y2, src, dst)
    h2, y3 = pl.pallas_call(
        _l2_body, out_shape=(_sds((N, 100)), _sds((N, 16))))(
            h1, g2[:N], g2[N:], dinv, W2_0, b2.reshape(1, -1), W3_1)

    g3 = _make_prop(16)(y3, src, dst)
    out = pl.pallas_call(
        _final_body, out_shape=_sds((N, 1)))(
            h2, g3[:N], g3[N:], dinv, W3_0, b3.reshape(1, -1))
    return out


# speedup vs baseline: 18.0701x; 1.6289x over previous
import functools

import jax
import jax.numpy as jnp
from jax import lax
from jax.experimental import pallas as pl
from jax.experimental.pallas import tpu as pltpu
from jax.experimental.pallas import tpu_sc as plsc

N = 10000
E = 160000
CHUNK = 128
NCHUNK = E // CHUNK
NW = 32
BASE_CH = NCHUNK // NW
EXTRA_CH = NCHUNK - BASE_CH * NW
RPT = 624


def _mesh():
    return plsc.VectorSubcoreMesh(core_axis_name="c", subcore_axis_name="s")


NBUF = 3
ITERS = BASE_CH // NBUF


@functools.cache
def _make_deg():
    @functools.partial(
        pl.kernel,
        out_type=jax.ShapeDtypeStruct((2 * N,), jnp.float32),
        mesh=_mesh(),
        scratch_types=[
            pltpu.VMEM((CHUNK,), jnp.int32),
            pltpu.VMEM((CHUNK,), jnp.int32),
            pltpu.VMEM((CHUNK,), jnp.int32),
            pltpu.VMEM((CHUNK,), jnp.float32),
            pltpu.VMEM((RPT,), jnp.float32),
            pltpu.VMEM_SHARED((N,), jnp.float32),
            pltpu.SemaphoreType.DMA,
            pltpu.SemaphoreType.DMA,
            pltpu.SemaphoreType.DMA,
            pltpu.SemaphoreType.DMA,
            pltpu.SemaphoreType.DMA,
            pltpu.SemaphoreType.DMA,
        ],
    )
    def deg_kernel(src_hbm, out_hbm, si0, si1, si2, ones_v, zbuf, acc,
                   is0, is1, is2, ss0, ss1, ss2):
        sidx = [si0, si1, si2]
        isem = [is0, is1, is2]
        ssem = [ss0, ss1, ss2]
        c = lax.axis_index("c")
        s = lax.axis_index("s")
        w = c * 16 + s
        one16 = jnp.full((16,), 1.0, jnp.float32)
        zero16 = jnp.zeros((16,), jnp.float32)
        for j in range(CHUNK // 16):
            ones_v[pl.ds(16 * j, 16)] = one16
        for j in range(RPT // 16):
            zbuf[pl.ds(16 * j, 16)] = zero16

        def idx_start(b, t):
            e0 = pl.multiple_of((w * BASE_CH + t) * CHUNK, CHUNK)
            pltpu.async_copy(src_hbm.at[pl.ds(e0, CHUNK)], sidx[b], isem[b])

        def idx_wait(b):
            pltpu.make_async_copy(
                src_hbm.at[pl.ds(0, CHUNK)], sidx[b], isem[b]).wait()

        for b in range(NBUF):
            idx_start(b, b)
        base = pl.multiple_of(s * RPT, 8)
        pltpu.sync_copy(zbuf, acc.at[pl.ds(base, RPT)])

        @pl.when(s == 15)
        def _():
            pltpu.sync_copy(zbuf.at[pl.ds(0, 16)], acc.at[pl.ds(N - 16, 16)])

        plsc.subcore_barrier()

        def giter(g, carry):
            for b in range(NBUF):
                idx_wait(b)
                pltpu.async_copy(ones_v, acc.at[sidx[b]], ssem[b], add=True)
            for b in range(NBUF):
                pltpu.make_async_copy(ones_v, acc.at[sidx[b]],
                                      ssem[b]).wait()

                @pl.when(g < ITERS - 1)
                def _():
                    idx_start(b, g * NBUF + NBUF + b)

            return carry

        lax.fori_loop(0, ITERS, giter, 0)

        @pl.when(w < EXTRA_CH)
        def _():
            e0x = pl.multiple_of((NW * BASE_CH + w) * CHUNK, CHUNK)
            pltpu.sync_copy(src_hbm.at[pl.ds(e0x, CHUNK)], sidx[0])
            pltpu.sync_copy(ones_v, acc.at[sidx[0]], add=True)

        plsc.subcore_barrier()
        ob = pl.multiple_of(c * N + base, 8)
        pltpu.sync_copy(acc.at[pl.ds(base, RPT)], zbuf)
        pltpu.sync_copy(zbuf, out_hbm.at[pl.ds(ob, RPT)])

        @pl.when(s == 15)
        def _():
            pltpu.sync_copy(acc.at[pl.ds(N - 16, 16)], zbuf.at[pl.ds(0, 16)])
            pltpu.sync_copy(zbuf.at[pl.ds(0, 16)],
                            out_hbm.at[pl.ds(c * N + N - 16, 16)])

    return deg_kernel


@functools.cache
def _make_prop(F):
    @functools.partial(
        pl.kernel,
        out_type=jax.ShapeDtypeStruct((2 * N, F), jnp.float32),
        mesh=_mesh(),
        scratch_types=[
            pltpu.VMEM((CHUNK,), jnp.int32),
            pltpu.VMEM((CHUNK,), jnp.int32),
            pltpu.VMEM((CHUNK,), jnp.int32),
            pltpu.VMEM((CHUNK,), jnp.int32),
            pltpu.VMEM((CHUNK,), jnp.int32),
            pltpu.VMEM((CHUNK,), jnp.int32),
            pltpu.VMEM((CHUNK, F), jnp.float32),
            pltpu.VMEM((CHUNK, F), jnp.float32),
            pltpu.VMEM((CHUNK, F), jnp.float32),
            pltpu.VMEM((16, F), jnp.float32),
            pltpu.VMEM_SHARED((N, F), jnp.float32),
            pltpu.SemaphoreType.DMA,
            pltpu.SemaphoreType.DMA,
            pltpu.SemaphoreType.DMA,
            pltpu.SemaphoreType.DMA,
            pltpu.SemaphoreType.DMA,
            pltpu.SemaphoreType.DMA,
            pltpu.SemaphoreType.DMA,
            pltpu.SemaphoreType.DMA,
            pltpu.SemaphoreType.DMA,
        ],
        compiler_params=pltpu.CompilerParams(use_tc_tiling_on_sc=False),
    )
    def prop_kernel(y_hbm, src_hbm, dst_hbm, out_hbm,
                    si0, si1, si2, di0, di1, di2, r0, r1, r2, zbuf, acc,
                    is0, is1, is2, gs0, gs1, gs2, ss0, ss1, ss2):
        sidx = [si0, si1, si2]
        didx = [di0, di1, di2]
        rows = [r0, r1, r2]
        isem = [is0, is1, is2]
        gsem = [gs0, gs1, gs2]
        ssem = [ss0, ss1, ss2]
        c = lax.axis_index("c")
        s = lax.axis_index("s")
        w = c * 16 + s
        zero16 = jnp.zeros((16,), jnp.float32)
        for r in range(16):
            for j in range(F // 16):
                zbuf[r, pl.ds(16 * j, 16)] = zero16

        def idx_start(b, t):
            e0 = pl.multiple_of((w * BASE_CH + t) * CHUNK, CHUNK)
            pltpu.async_copy(src_hbm.at[pl.ds(e0, CHUNK)], sidx[b], isem[b])
            pltpu.async_copy(dst_hbm.at[pl.ds(e0, CHUNK)], didx[b], isem[b])

        def idx_wait(b):
            pltpu.make_async_copy(
                src_hbm.at[pl.ds(0, CHUNK)], sidx[b], isem[b]).wait()
            pltpu.make_async_copy(
                dst_hbm.at[pl.ds(0, CHUNK)], didx[b], isem[b]).wait()

        for b in range(NBUF):
            idx_start(b, b)

        base = pl.multiple_of(s * RPT, 8)
        nz = RPT // 16
        for i in range(nz):
            pltpu.async_copy(zbuf, acc.at[pl.ds(base + 16 * i, 16)], gsem[0])

        @pl.when(s == 15)
        def _():
            pltpu.async_copy(zbuf, acc.at[pl.ds(N - 16, 16)], gsem[0])

        for i in range(nz):
            pltpu.make_async_copy(zbuf, acc.at[pl.ds(base, 16)],
                                  gsem[0]).wait()

        @pl.when(s == 15)
        def _():
            pltpu.make_async_copy(zbuf, acc.at[pl.ds(N - 16, 16)],
                                  gsem[0]).wait()

        plsc.subcore_barrier()

        def giter(g, carry):
            for b in range(NBUF):
                idx_wait(b)
                pltpu.async_copy(y_hbm.at[sidx[b]], rows[b], gsem[b])
            for b in range(NBUF):
                pltpu.make_async_copy(y_hbm.at[sidx[b]], rows[b],
                                      gsem[b]).wait()
                pltpu.async_copy(rows[b], acc.at[didx[b]], ssem[b], add=True)
            for b in range(NBUF):
                pltpu.make_async_copy(rows[b], acc.at[didx[b]],
                                      ssem[b]).wait()

                @pl.when(g < ITERS - 1)
                def _():
                    idx_start(b, g * NBUF + NBUF + b)

            return carry

        lax.fori_loop(0, ITERS, giter, 0)

        @pl.when(w < EXTRA_CH)
        def _():
            e0x = pl.multiple_of((NW * BASE_CH + w) * CHUNK, CHUNK)
            pltpu.sync_copy(src_hbm.at[pl.ds(e0x, CHUNK)], sidx[0])
            pltpu.sync_copy(dst_hbm.at[pl.ds(e0x, CHUNK)], didx[0])
            pltpu.sync_copy(y_hbm.at[sidx[0]], rows[0])
            pltpu.sync_copy(rows[0], acc.at[didx[0]], add=True)

        plsc.subcore_barrier()

        ob = pl.multiple_of(c * N + base, 8)
        plan = [(0, 128), (128, 128), (256, 128), (384, 128), (512, 112)]
        for i, (off, sz) in enumerate(plan):
            b = i % NBUF
            if i >= NBUF:
                poff, psz = plan[i - NBUF]
                pltpu.make_async_copy(
                    rows[b].at[pl.ds(0, psz)],
                    out_hbm.at[pl.ds(ob + poff, psz)], ssem[b]).wait()
            pltpu.async_copy(acc.at[pl.ds(base + off, sz)],
                             rows[b].at[pl.ds(0, sz)], gsem[b])
            pltpu.make_async_copy(acc.at[pl.ds(base + off, sz)],
                                  rows[b].at[pl.ds(0, sz)], gsem[b]).wait()
            pltpu.async_copy(rows[b].at[pl.ds(0, sz)],
                             out_hbm.at[pl.ds(ob + off, sz)], ssem[b])
        for i in range(len(plan) - NBUF, len(plan)):
            b = i % NBUF
            off, sz = plan[i]
            pltpu.make_async_copy(rows[b].at[pl.ds(0, sz)],
                                  out_hbm.at[pl.ds(ob + off, sz)],
                                  ssem[b]).wait()

        @pl.when(s == 15)
        def _():
            pltpu.sync_copy(acc.at[pl.ds(N - 16, 16)], zbuf)
            pltpu.sync_copy(zbuf, out_hbm.at[pl.ds(c * N + N - 16, 16)])

    return prop_kernel


def _prep_body(degp_ref, x_ref, dinv_ref, xs_ref):
    dp = degp_ref[...]
    ones2 = jnp.ones((2, 1), jnp.float32)
    deg = lax.dot_general(dp, ones2, (((0,), (0,)), ((), ())),
                          preferred_element_type=jnp.float32)
    safe = jnp.where(deg > 0, deg, 1.0)
    dinv = jnp.where(deg > 0, lax.rsqrt(safe), 0.0)
    dinv_ref[...] = dinv
    xs = x_ref[...] * dinv
    xs_ref[...] = jnp.concatenate(
        [xs, jnp.zeros((N, 64 - 58), jnp.float32)], axis=1)


def _l1_body(x_ref, g1a_ref, g1b_ref, dinv_ref, w10_ref, w11_ref, b1_ref,
             w21_ref, h1_ref, y2_ref):
    dinv = dinv_ref[...]
    tx1 = -(g1a_ref[...] + g1b_ref[...])[:, :58] * dinv
    h1 = jnp.dot(x_ref[...], w10_ref[...], preferred_element_type=jnp.float32)
    h1 += jnp.dot(tx1, w11_ref[...], preferred_element_type=jnp.float32)
    h1 = jnp.maximum(h1 + b1_ref[...], 0.0)
    h1_ref[...] = h1
    y2 = jnp.dot(h1, w21_ref[...], preferred_element_type=jnp.float32) * dinv
    y2_ref[...] = jnp.concatenate(
        [y2, jnp.zeros((N, 112 - 100), jnp.float32)], axis=1)


def _l2_body(h1_ref, g2a_ref, g2b_ref, dinv_ref, w20_ref, b2_ref, w31_ref,
             h2_ref, y3_ref):
    dinv = dinv_ref[...]
    tx2w = -(g2a_ref[...] + g2b_ref[...])[:, :100] * dinv
    h2 = jnp.dot(h1_ref[...], w20_ref[...],
                 preferred_element_type=jnp.float32)
    h2 = jnp.maximum(h2 + tx2w + b2_ref[...], 0.0)
    h2_ref[...] = h2
    y3 = jnp.dot(h2, w31_ref[...], preferred_element_type=jnp.float32) * dinv
    y3_ref[...] = jnp.concatenate(
        [y3, jnp.zeros((N, 15), jnp.float32)], axis=1)


def _final_body(h2_ref, g3a_ref, g3b_ref, dinv_ref, w30_ref, b3_ref, out_ref):
    tx3w = -(g3a_ref[...] + g3b_ref[...])[:, :1] * dinv_ref[...]
    out = jnp.dot(h2_ref[...], w30_ref[...],
                  preferred_element_type=jnp.float32)
    out_ref[...] = out + tx3w + b3_ref[...]


def _sds(shape):
    return jax.ShapeDtypeStruct(shape, jnp.float32)


def kernel(x, edge_index, W1_0, W1_1, b1, W2_0, W2_1, b2, W3_0, W3_1, b3):
    src = edge_index[0]
    dst = edge_index[1]

    degp = _make_deg()(src).reshape(2, N)

    dinv, xs = pl.pallas_call(
        _prep_body, out_shape=(_sds((N, 1)), _sds((N, 64))))(degp, x)

    g1 = _make_prop(64)(xs, src, dst)
    h1, y2 = pl.pallas_call(
        _l1_body, out_shape=(_sds((N, 300)), _sds((N, 112))))(
            x, g1[:N], g1[N:], dinv, W1_0, W1_1, b1.reshape(1, -1), W2_1)

    g2 = _make_prop(112)(y2, src, dst)
    h2, y3 = pl.pallas_call(
        _l2_body, out_shape=(_sds((N, 100)), _sds((N, 16))))(
            h1, g2[:N], g2[N:], dinv, W2_0, b2.reshape(1, -1), W3_1)

    g3 = _make_prop(16)(y3, src, dst)
    out = pl.pallas_call(
        _final_body, out_shape=_sds((N, 1)))(
            h2, g3[:N], g3[N:], dinv, W3_0, b3.reshape(1, -1))
    return out


# speedup vs baseline: 20.0156x; 1.1077x over previous
import functools

import jax
import jax.numpy as jnp
from jax import lax
from jax.experimental import pallas as pl
from jax.experimental.pallas import tpu as pltpu
from jax.experimental.pallas import tpu_sc as plsc

N = 10000
E = 160000
CHUNK = 128
NW = 32
RPT = 624


def _mesh():
    return plsc.VectorSubcoreMesh(core_axis_name="c", subcore_axis_name="s")


NBUF = 3


def _edge_split(chunk):
    nchunk = E // chunk
    base = nchunk // NW
    extra = nchunk - base * NW
    assert base % NBUF == 0 and (chunk * NW) % 8 == 0
    return base, extra, base // NBUF


BASE_CH, EXTRA_CH, ITERS = _edge_split(CHUNK)


def _wb_plan(chunk):
    plan, off = [], 0
    while off + chunk <= RPT:
        plan.append((off, chunk))
        off += chunk
    if off < RPT:
        plan.append((off, RPT - off))
    return plan


@functools.cache
def _make_deg():
    @functools.partial(
        pl.kernel,
        out_type=jax.ShapeDtypeStruct((2 * N,), jnp.float32),
        mesh=_mesh(),
        scratch_types=[
            pltpu.VMEM((CHUNK,), jnp.int32),
            pltpu.VMEM((CHUNK,), jnp.int32),
            pltpu.VMEM((CHUNK,), jnp.int32),
            pltpu.VMEM((CHUNK,), jnp.float32),
            pltpu.VMEM((RPT,), jnp.float32),
            pltpu.VMEM_SHARED((N,), jnp.float32),
            pltpu.SemaphoreType.DMA,
            pltpu.SemaphoreType.DMA,
            pltpu.SemaphoreType.DMA,
            pltpu.SemaphoreType.DMA,
            pltpu.SemaphoreType.DMA,
            pltpu.SemaphoreType.DMA,
        ],
    )
    def deg_kernel(src_hbm, out_hbm, si0, si1, si2, ones_v, zbuf, acc,
                   is0, is1, is2, ss0, ss1, ss2):
        sidx = [si0, si1, si2]
        isem = [is0, is1, is2]
        ssem = [ss0, ss1, ss2]
        c = lax.axis_index("c")
        s = lax.axis_index("s")
        w = c * 16 + s
        one16 = jnp.full((16,), 1.0, jnp.float32)
        zero16 = jnp.zeros((16,), jnp.float32)
        for j in range(CHUNK // 16):
            ones_v[pl.ds(16 * j, 16)] = one16
        for j in range(RPT // 16):
            zbuf[pl.ds(16 * j, 16)] = zero16

        def idx_start(b, t):
            e0 = pl.multiple_of((w * BASE_CH + t) * CHUNK, CHUNK)
            pltpu.async_copy(src_hbm.at[pl.ds(e0, CHUNK)], sidx[b], isem[b])

        def idx_wait(b):
            pltpu.make_async_copy(
                src_hbm.at[pl.ds(0, CHUNK)], sidx[b], isem[b]).wait()

        for b in range(NBUF):
            idx_start(b, b)
        base = pl.multiple_of(s * RPT, 8)
        pltpu.sync_copy(zbuf, acc.at[pl.ds(base, RPT)])

        @pl.when(s == 15)
        def _():
            pltpu.sync_copy(zbuf.at[pl.ds(0, 16)], acc.at[pl.ds(N - 16, 16)])

        plsc.subcore_barrier()

        def giter(g, carry):
            for b in range(NBUF):
                idx_wait(b)
                pltpu.async_copy(ones_v, acc.at[sidx[b]], ssem[b], add=True)
            for b in range(NBUF):
                pltpu.make_async_copy(ones_v, acc.at[sidx[b]],
                                      ssem[b]).wait()

                @pl.when(g < ITERS - 1)
                def _():
                    idx_start(b, g * NBUF + NBUF + b)

            return carry

        lax.fori_loop(0, ITERS, giter, 0)

        @pl.when(w < EXTRA_CH)
        def _():
            e0x = pl.multiple_of((NW * BASE_CH + w) * CHUNK, CHUNK)
            pltpu.sync_copy(src_hbm.at[pl.ds(e0x, CHUNK)], sidx[0])
            pltpu.sync_copy(ones_v, acc.at[sidx[0]], add=True)

        plsc.subcore_barrier()
        ob = pl.multiple_of(c * N + base, 8)
        pltpu.sync_copy(acc.at[pl.ds(base, RPT)], zbuf)
        pltpu.sync_copy(zbuf, out_hbm.at[pl.ds(ob, RPT)])

        @pl.when(s == 15)
        def _():
            pltpu.sync_copy(acc.at[pl.ds(N - 16, 16)], zbuf.at[pl.ds(0, 16)])
            pltpu.sync_copy(zbuf.at[pl.ds(0, 16)],
                            out_hbm.at[pl.ds(c * N + N - 16, 16)])

    return deg_kernel


@functools.cache
def _make_prop(F, chunk, tiled):
    base_ch, extra_ch, iters = _edge_split(chunk)
    cparams = None if tiled else pltpu.CompilerParams(
        use_tc_tiling_on_sc=False)

    @functools.partial(
        pl.kernel,
        out_type=jax.ShapeDtypeStruct((2 * N, F), jnp.float32),
        mesh=_mesh(),
        scratch_types=[
            pltpu.VMEM((chunk,), jnp.int32),
            pltpu.VMEM((chunk,), jnp.int32),
            pltpu.VMEM((chunk,), jnp.int32),
            pltpu.VMEM((chunk,), jnp.int32),
            pltpu.VMEM((chunk,), jnp.int32),
            pltpu.VMEM((chunk,), jnp.int32),
            pltpu.VMEM((chunk, F), jnp.float32),
            pltpu.VMEM((chunk, F), jnp.float32),
            pltpu.VMEM((chunk, F), jnp.float32),
            pltpu.VMEM((16, F), jnp.float32),
            pltpu.VMEM_SHARED((N, F), jnp.float32),
            pltpu.SemaphoreType.DMA,
            pltpu.SemaphoreType.DMA,
            pltpu.SemaphoreType.DMA,
            pltpu.SemaphoreType.DMA,
            pltpu.SemaphoreType.DMA,
            pltpu.SemaphoreType.DMA,
            pltpu.SemaphoreType.DMA,
            pltpu.SemaphoreType.DMA,
            pltpu.SemaphoreType.DMA,
        ],
        compiler_params=cparams,
    )
    def prop_kernel(y_hbm, src_hbm, dst_hbm, out_hbm,
                    si0, si1, si2, di0, di1, di2, r0, r1, r2, zbuf, acc,
                    is0, is1, is2, gs0, gs1, gs2, ss0, ss1, ss2):
        sidx = [si0, si1, si2]
        didx = [di0, di1, di2]
        rows = [r0, r1, r2]
        isem = [is0, is1, is2]
        gsem = [gs0, gs1, gs2]
        ssem = [ss0, ss1, ss2]
        c = lax.axis_index("c")
        s = lax.axis_index("s")
        w = c * 16 + s
        zero16 = jnp.zeros((16,), jnp.float32)
        for r in range(16):
            for j in range(F // 16):
                zbuf[r, pl.ds(16 * j, 16)] = zero16

        def idx_start(b, t):
            e0 = pl.multiple_of((w * base_ch + t) * chunk, 8)
            pltpu.async_copy(src_hbm.at[pl.ds(e0, chunk)], sidx[b], isem[b])
            pltpu.async_copy(dst_hbm.at[pl.ds(e0, chunk)], didx[b], isem[b])

        def idx_wait(b):
            pltpu.make_async_copy(
                src_hbm.at[pl.ds(0, chunk)], sidx[b], isem[b]).wait()
            pltpu.make_async_copy(
                dst_hbm.at[pl.ds(0, chunk)], didx[b], isem[b]).wait()

        for b in range(NBUF):
            idx_start(b, b)

        base = pl.multiple_of(s * RPT, 8)
        nz = RPT // 16
        for i in range(nz):
            pltpu.async_copy(zbuf, acc.at[pl.ds(base + 16 * i, 16)], gsem[0])

        @pl.when(s == 15)
        def _():
            pltpu.async_copy(zbuf, acc.at[pl.ds(N - 16, 16)], gsem[0])

        for i in range(nz):
            pltpu.make_async_copy(zbuf, acc.at[pl.ds(base, 16)],
                                  gsem[0]).wait()

        @pl.when(s == 15)
        def _():
            pltpu.make_async_copy(zbuf, acc.at[pl.ds(N - 16, 16)],
                                  gsem[0]).wait()

        plsc.subcore_barrier()

        def giter(g, carry):
            for b in range(NBUF):
                idx_wait(b)
                pltpu.async_copy(y_hbm.at[sidx[b]], rows[b], gsem[b])
            for b in range(NBUF):
                pltpu.make_async_copy(y_hbm.at[sidx[b]], rows[b],
                                      gsem[b]).wait()
                pltpu.async_copy(rows[b], acc.at[didx[b]], ssem[b], add=True)
            for b in range(NBUF):
                pltpu.make_async_copy(rows[b], acc.at[didx[b]],
                                      ssem[b]).wait()

                @pl.when(g < iters - 1)
                def _():
                    idx_start(b, g * NBUF + NBUF + b)

            return carry

        lax.fori_loop(0, iters, giter, 0)

        @pl.when(w < extra_ch)
        def _():
            e0x = pl.multiple_of((NW * base_ch + w) * chunk, 8)
            pltpu.sync_copy(src_hbm.at[pl.ds(e0x, chunk)], sidx[0])
            pltpu.sync_copy(dst_hbm.at[pl.ds(e0x, chunk)], didx[0])
            pltpu.sync_copy(y_hbm.at[sidx[0]], rows[0])
            pltpu.sync_copy(rows[0], acc.at[didx[0]], add=True)

        plsc.subcore_barrier()

        ob = pl.multiple_of(c * N + base, 8)
        plan = _wb_plan(chunk)
        for i, (off, sz) in enumerate(plan):
            b = i % NBUF
            if i >= NBUF:
                poff, psz = plan[i - NBUF]
                pltpu.make_async_copy(
                    rows[b].at[pl.ds(0, psz)],
                    out_hbm.at[pl.ds(ob + poff, psz)], ssem[b]).wait()
            pltpu.async_copy(acc.at[pl.ds(base + off, sz)],
                             rows[b].at[pl.ds(0, sz)], gsem[b])
            pltpu.make_async_copy(acc.at[pl.ds(base + off, sz)],
                                  rows[b].at[pl.ds(0, sz)], gsem[b]).wait()
            pltpu.async_copy(rows[b].at[pl.ds(0, sz)],
                             out_hbm.at[pl.ds(ob + off, sz)], ssem[b])
        for i in range(max(0, len(plan) - NBUF), len(plan)):
            b = i % NBUF
            off, sz = plan[i]
            pltpu.make_async_copy(rows[b].at[pl.ds(0, sz)],
                                  out_hbm.at[pl.ds(ob + off, sz)],
                                  ssem[b]).wait()

        @pl.when(s == 15)
        def _():
            pltpu.sync_copy(acc.at[pl.ds(N - 16, 16)], zbuf)
            pltpu.sync_copy(zbuf, out_hbm.at[pl.ds(c * N + N - 16, 16)])

    return prop_kernel


def _a1_body(x_ref, w10_ref, b1_ref, a1_ref):
    a1_ref[...] = jnp.dot(x_ref[...], w10_ref[...],
                          preferred_element_type=jnp.float32) + b1_ref[...]


def _prep_body(degp_ref, x_ref, dinv_ref, xs_ref):
    dp = degp_ref[...]
    ones2 = jnp.ones((2, 1), jnp.float32)
    deg = lax.dot_general(dp, ones2, (((0,), (0,)), ((), ())),
                          preferred_element_type=jnp.float32)
    safe = jnp.where(deg > 0, deg, 1.0)
    dinv = jnp.where(deg > 0, lax.rsqrt(safe), 0.0)
    dinv_ref[...] = dinv
    xs = x_ref[...] * dinv
    xs_ref[...] = jnp.concatenate(
        [xs, jnp.zeros((N, 64 - 58), jnp.float32)], axis=1)


BN1 = 2000


def _post1_body(a1_ref, g1_ref, dinv_ref, w11_ref, w21_ref, h1_ref, y2_ref):
    dinv = dinv_ref[...]
    g = g1_ref[...]
    tx1 = -(g[0] + g[1])[:, :58] * dinv
    h1 = jnp.maximum(
        a1_ref[...] + jnp.dot(tx1, w11_ref[...],
                              preferred_element_type=jnp.float32), 0.0)
    h1_ref[...] = h1
    y2 = jnp.dot(h1, w21_ref[...], preferred_element_type=jnp.float32) * dinv
    y2_ref[...] = jnp.concatenate(
        [y2, jnp.zeros((BN1, 128 - 100), jnp.float32)], axis=1)


def _a2_body(h1_ref, w20_ref, b2_ref, a2_ref):
    a2_ref[...] = jnp.dot(h1_ref[...], w20_ref[...],
                          preferred_element_type=jnp.float32) + b2_ref[...]


def _post2_body(a2_ref, g2_ref, dinv_ref, w31_ref, h2_ref, y3_ref):
    dinv = dinv_ref[...]
    g = g2_ref[...]
    tx2w = -(g[:N] + g[N:])[:, :100] * dinv
    h2 = jnp.maximum(a2_ref[...] + tx2w, 0.0)
    h2_ref[...] = h2
    y3 = jnp.dot(h2, w31_ref[...], preferred_element_type=jnp.float32) * dinv
    y3_ref[...] = jnp.concatenate(
        [y3, jnp.zeros((N, 15), jnp.float32)], axis=1)


def _a3_body(h2_ref, w30_ref, b3_ref, a3_ref):
    a3_ref[...] = jnp.dot(h2_ref[...], w30_ref[...],
                          preferred_element_type=jnp.float32) + b3_ref[...]


def _final_body(a3_ref, g3_ref, dinv_ref, out_ref):
    g = g3_ref[...]
    out_ref[...] = a3_ref[...] - (g[:N, :1] + g[N:, :1]) * dinv_ref[...]


def _sds(shape):
    return jax.ShapeDtypeStruct(shape, jnp.float32)


def kernel(x, edge_index, W1_0, W1_1, b1, W2_0, W2_1, b2, W3_0, W3_1, b3):
    src = edge_index[0]
    dst = edge_index[1]

    a1 = pl.pallas_call(_a1_body, out_shape=_sds((N, 300)))(
        x, W1_0, b1.reshape(1, -1))
    degp = _make_deg()(src).reshape(2, N)
    dinv, xs = pl.pallas_call(
        _prep_body, out_shape=(_sds((N, 1)), _sds((N, 64))))(degp, x)

    g1 = _make_prop(64, CHUNK, False)(xs, src, dst)
    h1, y2 = pl.pallas_call(
        _post1_body,
        grid=(N // BN1,),
        in_specs=[
            pl.BlockSpec((BN1, 300), lambda i: (i, 0)),
            pl.BlockSpec((2, BN1, 64), lambda i: (0, i, 0)),
            pl.BlockSpec((BN1, 1), lambda i: (i, 0)),
            pl.BlockSpec((58, 300), lambda i: (0, 0)),
            pl.BlockSpec((300, 100), lambda i: (0, 0)),
        ],
        out_specs=(pl.BlockSpec((BN1, 300), lambda i: (i, 0)),
                   pl.BlockSpec((BN1, 128), lambda i: (i, 0))),
        out_shape=(_sds((N, 300)), _sds((N, 128))))(
            a1, g1.reshape(2, N, 64), dinv, W1_1, W2_1)
    a2 = pl.pallas_call(_a2_body, out_shape=_sds((N, 100)))(
        h1, W2_0, b2.reshape(1, -1))

    g2 = _make_prop(128, 64, True)(y2, src, dst)
    h2, y3 = pl.pallas_call(
        _post2_body, out_shape=(_sds((N, 100)), _sds((N, 16))))(
            a2, g2, dinv, W3_1)
    a3 = pl.pallas_call(_a3_body, out_shape=_sds((N, 1)))(
        h2, W3_0, b3.reshape(1, -1))

    g3 = _make_prop(16, CHUNK, False)(y3, src, dst)
    out = pl.pallas_call(_final_body, out_shape=_sds((N, 1)))(a3, g3, dinv)
    return out


# speedup vs baseline: 20.0282x; 1.0006x over previous
import functools

import jax
import jax.numpy as jnp
from jax import lax
from jax.experimental import pallas as pl
from jax.experimental.pallas import tpu as pltpu
from jax.experimental.pallas import tpu_sc as plsc

N = 10000
E = 160000
CHUNK = 128
NW = 32
RPT = 624


def _mesh():
    return plsc.VectorSubcoreMesh(core_axis_name="c", subcore_axis_name="s")


NBUF = 3


def _edge_split(chunk):
    nchunk = E // chunk
    base = nchunk // NW
    extra = nchunk - base * NW
    assert base % NBUF == 0 and (chunk * NW) % 8 == 0
    return base, extra, base // NBUF


BASE_CH, EXTRA_CH, ITERS = _edge_split(CHUNK)


def _wb_plan(chunk):
    plan, off = [], 0
    while off + chunk <= RPT:
        plan.append((off, chunk))
        off += chunk
    if off < RPT:
        plan.append((off, RPT - off))
    return plan


@functools.cache
def _make_deg():
    @functools.partial(
        pl.kernel,
        out_type=jax.ShapeDtypeStruct((2 * N,), jnp.float32),
        mesh=_mesh(),
        scratch_types=[
            pltpu.VMEM((CHUNK,), jnp.int32),
            pltpu.VMEM((CHUNK,), jnp.int32),
            pltpu.VMEM((CHUNK,), jnp.int32),
            pltpu.VMEM((CHUNK,), jnp.float32),
            pltpu.VMEM((RPT,), jnp.float32),
            pltpu.VMEM_SHARED((N,), jnp.float32),
            pltpu.SemaphoreType.DMA,
            pltpu.SemaphoreType.DMA,
            pltpu.SemaphoreType.DMA,
            pltpu.SemaphoreType.DMA,
            pltpu.SemaphoreType.DMA,
            pltpu.SemaphoreType.DMA,
        ],
    )
    def deg_kernel(src_hbm, out_hbm, si0, si1, si2, ones_v, zbuf, acc,
                   is0, is1, is2, ss0, ss1, ss2):
        sidx = [si0, si1, si2]
        isem = [is0, is1, is2]
        ssem = [ss0, ss1, ss2]
        c = lax.axis_index("c")
        s = lax.axis_index("s")
        w = c * 16 + s
        one16 = jnp.full((16,), 1.0, jnp.float32)
        zero16 = jnp.zeros((16,), jnp.float32)
        for j in range(CHUNK // 16):
            ones_v[pl.ds(16 * j, 16)] = one16
        for j in range(RPT // 16):
            zbuf[pl.ds(16 * j, 16)] = zero16

        def idx_start(b, t):
            e0 = pl.multiple_of((w * BASE_CH + t) * CHUNK, CHUNK)
            pltpu.async_copy(src_hbm.at[pl.ds(e0, CHUNK)], sidx[b], isem[b])

        def idx_wait(b):
            pltpu.make_async_copy(
                src_hbm.at[pl.ds(0, CHUNK)], sidx[b], isem[b]).wait()

        for b in range(NBUF):
            idx_start(b, b)
        base = pl.multiple_of(s * RPT, 8)
        pltpu.sync_copy(zbuf, acc.at[pl.ds(base, RPT)])

        @pl.when(s == 15)
        def _():
            pltpu.sync_copy(zbuf.at[pl.ds(0, 16)], acc.at[pl.ds(N - 16, 16)])

        plsc.subcore_barrier()

        def giter(g, carry):
            for b in range(NBUF):
                idx_wait(b)
                pltpu.async_copy(ones_v, acc.at[sidx[b]], ssem[b], add=True)
            for b in range(NBUF):
                pltpu.make_async_copy(ones_v, acc.at[sidx[b]],
                                      ssem[b]).wait()

                @pl.when(g < ITERS - 1)
                def _():
                    idx_start(b, g * NBUF + NBUF + b)

            return carry

        lax.fori_loop(0, ITERS, giter, 0)

        @pl.when(w < EXTRA_CH)
        def _():
            e0x = pl.multiple_of((NW * BASE_CH + w) * CHUNK, CHUNK)
            pltpu.sync_copy(src_hbm.at[pl.ds(e0x, CHUNK)], sidx[0])
            pltpu.sync_copy(ones_v, acc.at[sidx[0]], add=True)

        plsc.subcore_barrier()
        ob = pl.multiple_of(c * N + base, 8)
        pltpu.sync_copy(acc.at[pl.ds(base, RPT)], zbuf)
        pltpu.sync_copy(zbuf, out_hbm.at[pl.ds(ob, RPT)])

        @pl.when(s == 15)
        def _():
            pltpu.sync_copy(acc.at[pl.ds(N - 16, 16)], zbuf.at[pl.ds(0, 16)])
            pltpu.sync_copy(zbuf.at[pl.ds(0, 16)],
                            out_hbm.at[pl.ds(c * N + N - 16, 16)])

    return deg_kernel


@functools.cache
def _make_prop(F, chunk, tiled):
    base_ch, extra_ch, iters = _edge_split(chunk)
    cparams = None if tiled else pltpu.CompilerParams(
        use_tc_tiling_on_sc=False)

    @functools.partial(
        pl.kernel,
        out_type=jax.ShapeDtypeStruct((2 * N, F), jnp.float32),
        mesh=_mesh(),
        scratch_types=[
            pltpu.VMEM((chunk,), jnp.int32),
            pltpu.VMEM((chunk,), jnp.int32),
            pltpu.VMEM((chunk,), jnp.int32),
            pltpu.VMEM((chunk,), jnp.int32),
            pltpu.VMEM((chunk,), jnp.int32),
            pltpu.VMEM((chunk,), jnp.int32),
            pltpu.VMEM((chunk, F), jnp.float32),
            pltpu.VMEM((chunk, F), jnp.float32),
            pltpu.VMEM((chunk, F), jnp.float32),
            pltpu.VMEM((16, F), jnp.float32),
            pltpu.VMEM_SHARED((N, F), jnp.float32),
            pltpu.SemaphoreType.DMA,
            pltpu.SemaphoreType.DMA,
            pltpu.SemaphoreType.DMA,
            pltpu.SemaphoreType.DMA,
            pltpu.SemaphoreType.DMA,
            pltpu.SemaphoreType.DMA,
            pltpu.SemaphoreType.DMA,
            pltpu.SemaphoreType.DMA,
            pltpu.SemaphoreType.DMA,
        ],
        compiler_params=cparams,
    )
    def prop_kernel(y_hbm, src_hbm, dst_hbm, out_hbm,
                    si0, si1, si2, di0, di1, di2, r0, r1, r2, zbuf, acc,
                    is0, is1, is2, gs0, gs1, gs2, ss0, ss1, ss2):
        sidx = [si0, si1, si2]
        didx = [di0, di1, di2]
        rows = [r0, r1, r2]
        isem = [is0, is1, is2]
        gsem = [gs0, gs1, gs2]
        ssem = [ss0, ss1, ss2]
        c = lax.axis_index("c")
        s = lax.axis_index("s")
        w = c * 16 + s
        zero16 = jnp.zeros((16,), jnp.float32)
        for r in range(16):
            for j in range(F // 16):
                zbuf[r, pl.ds(16 * j, 16)] = zero16

        def idx_start(b, t):
            e0 = pl.multiple_of((w * base_ch + t) * chunk, 8)
            pltpu.async_copy(src_hbm.at[pl.ds(e0, chunk)], sidx[b], isem[b])
            pltpu.async_copy(dst_hbm.at[pl.ds(e0, chunk)], didx[b], isem[b])

        def idx_wait(b):
            pltpu.make_async_copy(
                src_hbm.at[pl.ds(0, chunk)], sidx[b], isem[b]).wait()
            pltpu.make_async_copy(
                dst_hbm.at[pl.ds(0, chunk)], didx[b], isem[b]).wait()

        for b in range(NBUF):
            idx_start(b, b)

        base = pl.multiple_of(s * RPT, 8)
        nz = RPT // 16
        for i in range(nz):
            pltpu.async_copy(zbuf, acc.at[pl.ds(base + 16 * i, 16)], gsem[0])

        @pl.when(s == 15)
        def _():
            pltpu.async_copy(zbuf, acc.at[pl.ds(N - 16, 16)], gsem[0])

        for i in range(nz):
            pltpu.make_async_copy(zbuf, acc.at[pl.ds(base, 16)],
                                  gsem[0]).wait()

        @pl.when(s == 15)
        def _():
            pltpu.make_async_copy(zbuf, acc.at[pl.ds(N - 16, 16)],
                                  gsem[0]).wait()

        plsc.subcore_barrier()

        def giter(g, carry):
            for b in range(NBUF):
                idx_wait(b)
                pltpu.async_copy(y_hbm.at[sidx[b]], rows[b], gsem[b])
            for b in range(NBUF):
                pltpu.make_async_copy(y_hbm.at[sidx[b]], rows[b],
                                      gsem[b]).wait()
                pltpu.async_copy(rows[b], acc.at[didx[b]], ssem[b], add=True)
            for b in range(NBUF):
                pltpu.make_async_copy(rows[b], acc.at[didx[b]],
                                      ssem[b]).wait()

                @pl.when(g < iters - 1)
                def _():
                    idx_start(b, g * NBUF + NBUF + b)

            return carry

        lax.fori_loop(0, iters, giter, 0)

        @pl.when(w < extra_ch)
        def _():
            e0x = pl.multiple_of((NW * base_ch + w) * chunk, 8)
            pltpu.sync_copy(src_hbm.at[pl.ds(e0x, chunk)], sidx[0])
            pltpu.sync_copy(dst_hbm.at[pl.ds(e0x, chunk)], didx[0])
            pltpu.sync_copy(y_hbm.at[sidx[0]], rows[0])
            pltpu.sync_copy(rows[0], acc.at[didx[0]], add=True)

        plsc.subcore_barrier()

        ob = pl.multiple_of(c * N + base, 8)
        plan = _wb_plan(chunk)
        for i, (off, sz) in enumerate(plan):
            b = i % NBUF
            if i >= NBUF:
                poff, psz = plan[i - NBUF]
                pltpu.make_async_copy(
                    rows[b].at[pl.ds(0, psz)],
                    out_hbm.at[pl.ds(ob + poff, psz)], ssem[b]).wait()
            pltpu.async_copy(acc.at[pl.ds(base + off, sz)],
                             rows[b].at[pl.ds(0, sz)], gsem[b])
            pltpu.make_async_copy(acc.at[pl.ds(base + off, sz)],
                                  rows[b].at[pl.ds(0, sz)], gsem[b]).wait()
            pltpu.async_copy(rows[b].at[pl.ds(0, sz)],
                             out_hbm.at[pl.ds(ob + off, sz)], ssem[b])
        for i in range(max(0, len(plan) - NBUF), len(plan)):
            b = i % NBUF
            off, sz = plan[i]
            pltpu.make_async_copy(rows[b].at[pl.ds(0, sz)],
                                  out_hbm.at[pl.ds(ob + off, sz)],
                                  ssem[b]).wait()

        @pl.when(s == 15)
        def _():
            pltpu.sync_copy(acc.at[pl.ds(N - 16, 16)], zbuf)
            pltpu.sync_copy(zbuf, out_hbm.at[pl.ds(c * N + N - 16, 16)])

    return prop_kernel


def _prep_body(degp_ref, x_ref, w10_ref, b1_ref, a1_ref, dinv_ref, xs_ref):
    a1_ref[...] = jnp.dot(x_ref[...], w10_ref[...],
                          preferred_element_type=jnp.float32) + b1_ref[...]
    dp = degp_ref[...]
    ones2 = jnp.ones((2, 1), jnp.float32)
    deg = lax.dot_general(dp, ones2, (((0,), (0,)), ((), ())),
                          preferred_element_type=jnp.float32)
    safe = jnp.where(deg > 0, deg, 1.0)
    dinv = jnp.where(deg > 0, lax.rsqrt(safe), 0.0)
    dinv_ref[...] = dinv
    xs = x_ref[...] * dinv
    xs_ref[...] = jnp.concatenate(
        [xs, jnp.zeros((N, 64 - 58), jnp.float32)], axis=1)


BN1 = 2000


def _post1_body(a1_ref, g1_ref, dinv_ref, w11_ref, w21_ref, w20_ref, b2_ref,
                y2_ref, a2_ref):
    dinv = dinv_ref[...]
    g = g1_ref[...]
    tx1 = -(g[0] + g[1])[:, :58] * dinv
    h1 = jnp.maximum(
        a1_ref[...] + jnp.dot(tx1, w11_ref[...],
                              preferred_element_type=jnp.float32), 0.0)
    y2 = jnp.dot(h1, w21_ref[...], preferred_element_type=jnp.float32) * dinv
    y2_ref[...] = jnp.concatenate(
        [y2, jnp.zeros((BN1, 128 - 100), jnp.float32)], axis=1)
    a2_ref[...] = jnp.dot(h1, w20_ref[...],
                          preferred_element_type=jnp.float32) + b2_ref[...]


def _post2_body(a2_ref, g2_ref, dinv_ref, w31_ref, w30_ref, b3_ref,
                y3_ref, a3_ref):
    dinv = dinv_ref[...]
    g = g2_ref[...]
    tx2w = -(g[:N] + g[N:])[:, :100] * dinv
    h2 = jnp.maximum(a2_ref[...] + tx2w, 0.0)
    y3 = jnp.dot(h2, w31_ref[...], preferred_element_type=jnp.float32) * dinv
    y3_ref[...] = jnp.concatenate(
        [y3, jnp.zeros((N, 15), jnp.float32)], axis=1)
    a3_ref[...] = jnp.dot(h2, w30_ref[...],
                          preferred_element_type=jnp.float32) + b3_ref[...]


def _final_body(a3_ref, g3_ref, dinv_ref, out_ref):
    g = g3_ref[...]
    out_ref[...] = a3_ref[...] - (g[:N, :1] + g[N:, :1]) * dinv_ref[...]


def _sds(shape):
    return jax.ShapeDtypeStruct(shape, jnp.float32)


def kernel(x, edge_index, W1_0, W1_1, b1, W2_0, W2_1, b2, W3_0, W3_1, b3):
    src = edge_index[0]
    dst = edge_index[1]

    degp = _make_deg()(src).reshape(2, N)
    a1, dinv, xs = pl.pallas_call(
        _prep_body,
        out_shape=(_sds((N, 300)), _sds((N, 1)), _sds((N, 64))))(
            degp, x, W1_0, b1.reshape(1, -1))

    g1 = _make_prop(64, CHUNK, False)(xs, src, dst)
    y2, a2 = pl.pallas_call(
        _post1_body,
        grid=(N // BN1,),
        in_specs=[
            pl.BlockSpec((BN1, 300), lambda i: (i, 0)),
            pl.BlockSpec((2, BN1, 64), lambda i: (0, i, 0)),
            pl.BlockSpec((BN1, 1), lambda i: (i, 0)),
            pl.BlockSpec((58, 300), lambda i: (0, 0)),
            pl.BlockSpec((300, 100), lambda i: (0, 0)),
            pl.BlockSpec((300, 100), lambda i: (0, 0)),
            pl.BlockSpec((1, 100), lambda i: (0, 0)),
        ],
        out_specs=(pl.BlockSpec((BN1, 128), lambda i: (i, 0)),
                   pl.BlockSpec((BN1, 100), lambda i: (i, 0))),
        out_shape=(_sds((N, 128)), _sds((N, 100))))(
            a1, g1.reshape(2, N, 64), dinv, W1_1, W2_1, W2_0,
            b2.reshape(1, -1))

    g2 = _make_prop(128, 64, True)(y2, src, dst)
    y3, a3 = pl.pallas_call(
        _post2_body,
        out_shape=(_sds((N, 16)), _sds((N, 1))))(
            a2, g2, dinv, W3_1, W3_0, b3.reshape(1, -1))

    g3 = _make_prop(16, CHUNK, False)(y3, src, dst)
    out = pl.pallas_call(_final_body, out_shape=_sds((N, 1)))(a3, g3, dinv)
    return out


# speedup vs baseline: 20.9721x; 1.0471x over previous
import functools

import jax
import jax.numpy as jnp
from jax import lax
from jax.experimental import pallas as pl
from jax.experimental.pallas import tpu as pltpu
from jax.experimental.pallas import tpu_sc as plsc

N = 10000
E = 160000
CHUNK = 128
NW = 32
RPT = 624


def _mesh():
    return plsc.VectorSubcoreMesh(core_axis_name="c", subcore_axis_name="s")


NBUF = 3


def _edge_split(chunk):
    nchunk = E // chunk
    base = nchunk // NW
    extra = nchunk - base * NW
    assert base % NBUF == 0 and (chunk * NW) % 8 == 0
    return base, extra, base // NBUF


BASE_CH, EXTRA_CH, ITERS = _edge_split(CHUNK)


def _wb_plan(chunk):
    plan, off = [], 0
    while off + chunk <= RPT:
        plan.append((off, chunk))
        off += chunk
    if off < RPT:
        plan.append((off, RPT - off))
    return plan


@functools.cache
def _make_deg():
    @functools.partial(
        pl.kernel,
        out_type=jax.ShapeDtypeStruct((2 * N,), jnp.float32),
        mesh=_mesh(),
        scratch_types=[
            pltpu.VMEM((CHUNK,), jnp.int32),
            pltpu.VMEM((CHUNK,), jnp.int32),
            pltpu.VMEM((CHUNK,), jnp.int32),
            pltpu.VMEM((CHUNK,), jnp.float32),
            pltpu.VMEM((RPT,), jnp.float32),
            pltpu.VMEM_SHARED((N,), jnp.float32),
            pltpu.SemaphoreType.DMA,
            pltpu.SemaphoreType.DMA,
            pltpu.SemaphoreType.DMA,
            pltpu.SemaphoreType.DMA,
            pltpu.SemaphoreType.DMA,
            pltpu.SemaphoreType.DMA,
        ],
        compiler_params=pltpu.CompilerParams(use_tc_tiling_on_sc=False),
    )
    def deg_kernel(ei_hbm, out_hbm, si0, si1, si2, ones_v, zbuf, acc,
                   is0, is1, is2, ss0, ss1, ss2):
        sidx = [si0, si1, si2]
        isem = [is0, is1, is2]
        ssem = [ss0, ss1, ss2]
        c = lax.axis_index("c")
        s = lax.axis_index("s")
        w = c * 16 + s
        one16 = jnp.full((16,), 1.0, jnp.float32)
        zero16 = jnp.zeros((16,), jnp.float32)
        for j in range(CHUNK // 16):
            ones_v[pl.ds(16 * j, 16)] = one16
        for j in range(RPT // 16):
            zbuf[pl.ds(16 * j, 16)] = zero16

        def idx_start(b, t):
            e0 = pl.multiple_of((w * BASE_CH + t) * CHUNK, CHUNK)
            pltpu.async_copy(ei_hbm.at[0, pl.ds(e0, CHUNK)], sidx[b], isem[b])

        def idx_wait(b):
            pltpu.make_async_copy(
                ei_hbm.at[0, pl.ds(0, CHUNK)], sidx[b], isem[b]).wait()

        for b in range(NBUF):
            idx_start(b, b)
        base = pl.multiple_of(s * RPT, 8)
        pltpu.sync_copy(zbuf, acc.at[pl.ds(base, RPT)])

        @pl.when(s == 15)
        def _():
            pltpu.sync_copy(zbuf.at[pl.ds(0, 16)], acc.at[pl.ds(N - 16, 16)])

        plsc.subcore_barrier()

        def giter(g, carry):
            for b in range(NBUF):
                idx_wait(b)
                pltpu.async_copy(ones_v, acc.at[sidx[b]], ssem[b], add=True)
            for b in range(NBUF):
                pltpu.make_async_copy(ones_v, acc.at[sidx[b]],
                                      ssem[b]).wait()

                @pl.when(g < ITERS - 1)
                def _():
                    idx_start(b, g * NBUF + NBUF + b)

            return carry

        lax.fori_loop(0, ITERS, giter, 0)

        @pl.when(w < EXTRA_CH)
        def _():
            e0x = pl.multiple_of((NW * BASE_CH + w) * CHUNK, CHUNK)
            pltpu.sync_copy(ei_hbm.at[0, pl.ds(e0x, CHUNK)], sidx[0])
            pltpu.sync_copy(ones_v, acc.at[sidx[0]], add=True)

        plsc.subcore_barrier()
        ob = pl.multiple_of(c * N + base, 8)
        pltpu.sync_copy(acc.at[pl.ds(base, RPT)], zbuf)
        pltpu.sync_copy(zbuf, out_hbm.at[pl.ds(ob, RPT)])

        @pl.when(s == 15)
        def _():
            pltpu.sync_copy(acc.at[pl.ds(N - 16, 16)], zbuf.at[pl.ds(0, 16)])
            pltpu.sync_copy(zbuf.at[pl.ds(0, 16)],
                            out_hbm.at[pl.ds(c * N + N - 16, 16)])

    return deg_kernel


@functools.cache
def _make_prop(F, chunk, tiled):
    base_ch, extra_ch, iters = _edge_split(chunk)
    cparams = None if tiled else pltpu.CompilerParams(
        use_tc_tiling_on_sc=False)

    def wrap(body):
        if tiled:
            def tiled_entry(y_hbm, src_hbm, dst_hbm, out_hbm, *rest):
                body(y_hbm,
                     lambda sl: src_hbm.at[sl],
                     lambda sl: dst_hbm.at[sl],
                     out_hbm, *rest)
            return tiled_entry

        def ei_entry(y_hbm, ei_hbm, out_hbm, *rest):
            body(y_hbm,
                 lambda sl: ei_hbm.at[0, sl],
                 lambda sl: ei_hbm.at[1, sl],
                 out_hbm, *rest)
        return ei_entry

    @functools.partial(
        pl.kernel,
        out_type=jax.ShapeDtypeStruct((2 * N, F), jnp.float32),
        mesh=_mesh(),
        scratch_types=[
            pltpu.VMEM((chunk,), jnp.int32),
            pltpu.VMEM((chunk,), jnp.int32),
            pltpu.VMEM((chunk,), jnp.int32),
            pltpu.VMEM((chunk,), jnp.int32),
            pltpu.VMEM((chunk,), jnp.int32),
            pltpu.VMEM((chunk,), jnp.int32),
            pltpu.VMEM((chunk, F), jnp.float32),
            pltpu.VMEM((chunk, F), jnp.float32),
            pltpu.VMEM((chunk, F), jnp.float32),
            pltpu.VMEM((16, F), jnp.float32),
            pltpu.VMEM_SHARED((N, F), jnp.float32),
            pltpu.SemaphoreType.DMA,
            pltpu.SemaphoreType.DMA,
            pltpu.SemaphoreType.DMA,
            pltpu.SemaphoreType.DMA,
            pltpu.SemaphoreType.DMA,
            pltpu.SemaphoreType.DMA,
            pltpu.SemaphoreType.DMA,
            pltpu.SemaphoreType.DMA,
            pltpu.SemaphoreType.DMA,
        ],
        compiler_params=cparams,
    )
    @wrap
    def prop_kernel(y_hbm, src_at, dst_at, out_hbm,
                    si0, si1, si2, di0, di1, di2, r0, r1, r2, zbuf, acc,
                    is0, is1, is2, gs0, gs1, gs2, ss0, ss1, ss2):
        sidx = [si0, si1, si2]
        didx = [di0, di1, di2]
        rows = [r0, r1, r2]
        isem = [is0, is1, is2]
        gsem = [gs0, gs1, gs2]
        ssem = [ss0, ss1, ss2]
        c = lax.axis_index("c")
        s = lax.axis_index("s")
        w = c * 16 + s
        zero16 = jnp.zeros((16,), jnp.float32)
        for r in range(16):
            for j in range(F // 16):
                zbuf[r, pl.ds(16 * j, 16)] = zero16

        def idx_start(b, t):
            e0 = pl.multiple_of((w * base_ch + t) * chunk, 8)
            pltpu.async_copy(src_at(pl.ds(e0, chunk)), sidx[b], isem[b])
            pltpu.async_copy(dst_at(pl.ds(e0, chunk)), didx[b], isem[b])

        def idx_wait(b):
            pltpu.make_async_copy(
                src_at(pl.ds(0, chunk)), sidx[b], isem[b]).wait()
            pltpu.make_async_copy(
                dst_at(pl.ds(0, chunk)), didx[b], isem[b]).wait()

        for b in range(NBUF):
            idx_start(b, b)

        base = pl.multiple_of(s * RPT, 8)
        nz = RPT // 16
        for i in range(nz):
            pltpu.async_copy(zbuf, acc.at[pl.ds(base + 16 * i, 16)], gsem[0])

        @pl.when(s == 15)
        def _():
            pltpu.async_copy(zbuf, acc.at[pl.ds(N - 16, 16)], gsem[0])

        for i in range(nz):
            pltpu.make_async_copy(zbuf, acc.at[pl.ds(base, 16)],
                                  gsem[0]).wait()

        @pl.when(s == 15)
        def _():
            pltpu.make_async_copy(zbuf, acc.at[pl.ds(N - 16, 16)],
                                  gsem[0]).wait()

        plsc.subcore_barrier()

        def giter(g, carry):
            for b in range(NBUF):
                idx_wait(b)
                pltpu.async_copy(y_hbm.at[sidx[b]], rows[b], gsem[b])
            for b in range(NBUF):
                pltpu.make_async_copy(y_hbm.at[sidx[b]], rows[b],
                                      gsem[b]).wait()
                pltpu.async_copy(rows[b], acc.at[didx[b]], ssem[b], add=True)
            for b in range(NBUF):
                pltpu.make_async_copy(rows[b], acc.at[didx[b]],
                                      ssem[b]).wait()

                @pl.when(g < iters - 1)
                def _():
                    idx_start(b, g * NBUF + NBUF + b)

            return carry

        lax.fori_loop(0, iters, giter, 0)

        @pl.when(w < extra_ch)
        def _():
            e0x = pl.multiple_of((NW * base_ch + w) * chunk, 8)
            pltpu.sync_copy(src_at(pl.ds(e0x, chunk)), sidx[0])
            pltpu.sync_copy(dst_at(pl.ds(e0x, chunk)), didx[0])
            pltpu.sync_copy(y_hbm.at[sidx[0]], rows[0])
            pltpu.sync_copy(rows[0], acc.at[didx[0]], add=True)

        plsc.subcore_barrier()

        ob = pl.multiple_of(c * N + base, 8)
        plan = _wb_plan(chunk)
        for i, (off, sz) in enumerate(plan):
            b = i % NBUF
            if i >= NBUF:
                poff, psz = plan[i - NBUF]
                pltpu.make_async_copy(
                    rows[b].at[pl.ds(0, psz)],
                    out_hbm.at[pl.ds(ob + poff, psz)], ssem[b]).wait()
            pltpu.async_copy(acc.at[pl.ds(base + off, sz)],
                             rows[b].at[pl.ds(0, sz)], gsem[b])
            pltpu.make_async_copy(acc.at[pl.ds(base + off, sz)],
                                  rows[b].at[pl.ds(0, sz)], gsem[b]).wait()
            pltpu.async_copy(rows[b].at[pl.ds(0, sz)],
                             out_hbm.at[pl.ds(ob + off, sz)], ssem[b])
        for i in range(max(0, len(plan) - NBUF), len(plan)):
            b = i % NBUF
            off, sz = plan[i]
            pltpu.make_async_copy(rows[b].at[pl.ds(0, sz)],
                                  out_hbm.at[pl.ds(ob + off, sz)],
                                  ssem[b]).wait()

        @pl.when(s == 15)
        def _():
            pltpu.sync_copy(acc.at[pl.ds(N - 16, 16)], zbuf)
            pltpu.sync_copy(zbuf, out_hbm.at[pl.ds(c * N + N - 16, 16)])

    return prop_kernel


def _prep_body(degp_ref, x_ref, dinv_ref, xs_ref):
    dp = degp_ref[...]
    ones2 = jnp.ones((2, 1), jnp.float32)
    deg = lax.dot_general(dp, ones2, (((0,), (0,)), ((), ())),
                          preferred_element_type=jnp.float32)
    safe = jnp.where(deg > 0, deg, 1.0)
    dinv = jnp.where(deg > 0, lax.rsqrt(safe), 0.0)
    dinv_ref[...] = dinv
    xs = x_ref[...] * dinv
    xs_ref[...] = jnp.concatenate(
        [xs, jnp.zeros((N, 64 - 58), jnp.float32)], axis=1)


BN1 = 2000


def _post1_body(x_ref, g1a_ref, g1b_ref, dinv_ref, w10_ref, b1_ref,
                w11_ref, w21_ref, w20_ref, b2_ref, y2_ref, a2_ref):
    dinv = dinv_ref[...]
    tx1 = -(g1a_ref[...] + g1b_ref[...])[:, :58] * dinv
    a1 = jnp.dot(x_ref[...], w10_ref[...],
                 preferred_element_type=jnp.float32) + b1_ref[...]
    h1 = jnp.maximum(
        a1 + jnp.dot(tx1, w11_ref[...],
                     preferred_element_type=jnp.float32), 0.0)
    y2 = jnp.dot(h1, w21_ref[...], preferred_element_type=jnp.float32) * dinv
    y2_ref[...] = jnp.concatenate(
        [y2, jnp.zeros((BN1, 128 - 100), jnp.float32)], axis=1)
    a2_ref[...] = jnp.dot(h1, w20_ref[...],
                          preferred_element_type=jnp.float32) + b2_ref[...]


def _post2_body(a2_ref, g2_ref, dinv_ref, w31_ref, w30_ref, b3_ref,
                y3_ref, a3_ref):
    dinv = dinv_ref[...]
    g = g2_ref[...]
    tx2w = -(g[:N] + g[N:])[:, :100] * dinv
    h2 = jnp.maximum(a2_ref[...] + tx2w, 0.0)
    y3 = jnp.dot(h2, w31_ref[...], preferred_element_type=jnp.float32) * dinv
    y3_ref[...] = jnp.concatenate(
        [y3, jnp.zeros((N, 15), jnp.float32)], axis=1)
    a3_ref[...] = jnp.dot(h2, w30_ref[...],
                          preferred_element_type=jnp.float32) + b3_ref[...]


def _final_body(a3_ref, g3_ref, dinv_ref, out_ref):
    g = g3_ref[...]
    out_ref[...] = a3_ref[...] - (g[:N, :1] + g[N:, :1]) * dinv_ref[...]


def _sds(shape):
    return jax.ShapeDtypeStruct(shape, jnp.float32)


def kernel(x, edge_index, W1_0, W1_1, b1, W2_0, W2_1, b2, W3_0, W3_1, b3):
    src = edge_index[0]
    dst = edge_index[1]

    degp = _make_deg()(edge_index).reshape(2, N)
    dinv, xs = pl.pallas_call(
        _prep_body,
        out_shape=(_sds((N, 1)), _sds((N, 64))))(degp, x)

    g1 = _make_prop(64, CHUNK, False)(xs, edge_index)
    nb1 = N // BN1
    y2, a2 = pl.pallas_call(
        _post1_body,
        grid=(nb1,),
        in_specs=[
            pl.BlockSpec((BN1, 58), lambda i: (i, 0)),
            pl.BlockSpec((BN1, 64), lambda i: (i, 0)),
            pl.BlockSpec((BN1, 64), lambda i: (i + nb1, 0)),
            pl.BlockSpec((BN1, 1), lambda i: (i, 0)),
            pl.BlockSpec((58, 300), lambda i: (0, 0)),
            pl.BlockSpec((1, 300), lambda i: (0, 0)),
            pl.BlockSpec((58, 300), lambda i: (0, 0)),
            pl.BlockSpec((300, 100), lambda i: (0, 0)),
            pl.BlockSpec((300, 100), lambda i: (0, 0)),
            pl.BlockSpec((1, 100), lambda i: (0, 0)),
        ],
        out_specs=(pl.BlockSpec((BN1, 128), lambda i: (i, 0)),
                   pl.BlockSpec((BN1, 100), lambda i: (i, 0))),
        out_shape=(_sds((N, 128)), _sds((N, 100))))(
            x, g1, g1, dinv, W1_0, b1.reshape(1, -1), W1_1, W2_1, W2_0,
            b2.reshape(1, -1))

    g2 = _make_prop(128, 64, True)(y2, src, dst)
    y3, a3 = pl.pallas_call(
        _post2_body,
        out_shape=(_sds((N, 16)), _sds((N, 1))))(
            a2, g2, dinv, W3_1, W3_0, b3.reshape(1, -1))

    g3 = _make_prop(16, CHUNK, False)(y3, edge_index)
    out = pl.pallas_call(_final_body, out_shape=_sds((N, 1)))(a3, g3, dinv)
    return out
